# Initial kernel scaffold; baseline (speedup 1.0000x reference)
import jax
import jax.numpy as jnp
from jax.experimental import pallas as pl


def kernel(x, edge_index, W1, b1, g1, be1, W2, b2, g2, be2, W3, b3, g3, be3, Wl1, bl1, Wl2, bl2):
    raise NotImplementedError("write your pallas kernel here")


# speedup vs baseline: 21.8832x; 21.8832x over previous
import functools

import jax
import jax.numpy as jnp
import numpy as np
from jax import lax
from jax.experimental import pallas as pl
from jax.experimental.pallas import tpu as pltpu
from jax.experimental.pallas import tpu_sc as plsc

_EPS = 1e-5

_NC = 2
_NS = 16
_NW = _NC * _NS
_CH = 16


def _pad_sizes(n, e):
  npad = ((n + 256 + 2047) // 2048) * 2048
  rows = -(-e // 128)
  rows = -(-rows // (_NW * _CH)) * (_NW * _CH)
  return npad, rows


def _row_block(r):
  return 56 if r % 56 == 0 else 16


def _mesh():
  return plsc.VectorSubcoreMesh(core_axis_name="c", subcore_axis_name="s",
                                num_cores=_NC, num_subcores=_NS)


def _sc_degree(dst2d, npad, rows):
  sl = npad // _NS
  rpw = rows // _NW
  chunks = rpw // _CH

  @functools.partial(
      pl.kernel,
      out_type=jax.ShapeDtypeStruct((_NW * sl,), jnp.float32),
      mesh=_mesh(),
      scratch_types=[
          pltpu.VMEM((_CH, 128), jnp.int32),
          pltpu.VMEM((128,), jnp.float32),
          pltpu.VMEM((sl,), jnp.float32),
          pltpu.VMEM_SHARED((npad,), jnp.float32),
          pltpu.SemaphoreType.DMA,
      ],
  )
  def k(dst_hbm, out_hbm, idx_v, ones_v, slice_v, acc_sp, sem):
    c = lax.axis_index("c")
    s = lax.axis_index("s")
    w = c * _NS + s

    @pl.loop(0, 8)
    def _(i):
      ones_v[pl.ds(i * 16, 16)] = jnp.ones((16,), jnp.float32)

    @pl.loop(0, sl // 16)
    def _(i):
      slice_v[pl.ds(i * 16, 16)] = jnp.zeros((16,), jnp.float32)

    pltpu.sync_copy(slice_v, acc_sp.at[pl.ds(s * sl, sl)])
    plsc.subcore_barrier()

    @pl.loop(0, chunks)
    def _(t):
      base = w * rpw + t * _CH
      pltpu.sync_copy(dst_hbm.at[pl.ds(base, _CH)], idx_v)
      ds = []
      for j in range(_CH):
        ds.append(pltpu.async_copy(ones_v, acc_sp.at[idx_v.at[j]], sem,
                                   add=True))
      for d in ds:
        d.wait()

    plsc.subcore_barrier()
    pltpu.sync_copy(acc_sp.at[pl.ds(s * sl, sl)], slice_v)
    pltpu.sync_copy(slice_v, out_hbm.at[pl.ds(w * sl, sl)])

  return k(dst2d)


def _sc_scalar_pass(src2d, dst2d, y_hbm, dinv_hbm, npad, rows):
  sl = npad // _NS
  rpw = rows // _NW
  chunks = rpw // _CH

  @functools.partial(
      pl.kernel,
      out_type=[
          jax.ShapeDtypeStruct((_NW * sl,), jnp.float32),
          jax.ShapeDtypeStruct((_NW * sl,), jnp.float32),
      ],
      mesh=_mesh(),
      scratch_types=[
          pltpu.VMEM((_CH, 128), jnp.int32),
          pltpu.VMEM((_CH, 128), jnp.int32),
          pltpu.VMEM((_CH, 128), jnp.float32),
          pltpu.VMEM((_CH, 128), jnp.float32),
          pltpu.VMEM((sl,), jnp.float32),
          pltpu.VMEM_SHARED((npad,), jnp.float32),
          pltpu.VMEM_SHARED((npad,), jnp.float32),
          pltpu.VMEM_SHARED((npad,), jnp.float32),
          pltpu.VMEM_SHARED((npad,), jnp.float32),
          pltpu.SemaphoreType.DMA,
          pltpu.SemaphoreType.DMA,
      ],
  )
  def k(src_hbm, dst_hbm, y_in, dinv_in, r1_out, cs_out,
        idx_s, idx_d, yv, dv, slice_v, y_sp, dinv_sp, r1_sp, cs_sp,
        gsem, ssem):
    c = lax.axis_index("c")
    s = lax.axis_index("s")
    w = c * _NS + s

    pltpu.sync_copy(y_in.at[pl.ds(s * sl, sl)], slice_v)
    pltpu.sync_copy(slice_v, y_sp.at[pl.ds(s * sl, sl)])
    pltpu.sync_copy(dinv_in.at[pl.ds(s * sl, sl)], slice_v)
    pltpu.sync_copy(slice_v, dinv_sp.at[pl.ds(s * sl, sl)])

    @pl.loop(0, sl // 16)
    def _(i):
      slice_v[pl.ds(i * 16, 16)] = jnp.zeros((16,), jnp.float32)

    pltpu.sync_copy(slice_v, r1_sp.at[pl.ds(s * sl, sl)])
    pltpu.sync_copy(slice_v, cs_sp.at[pl.ds(s * sl, sl)])
    plsc.subcore_barrier()

    @pl.loop(0, chunks)
    def _(t):
      base = w * rpw + t * _CH
      pltpu.sync_copy(src_hbm.at[pl.ds(base, _CH)], idx_s)
      pltpu.sync_copy(dst_hbm.at[pl.ds(base, _CH)], idx_d)
      gs = []
      for j in range(_CH):
        gs.append(pltpu.async_copy(y_sp.at[idx_s.at[j]], yv.at[j], gsem))
        gs.append(pltpu.async_copy(dinv_sp.at[idx_d.at[j]], dv.at[j], gsem))
      for d in gs:
        d.wait()
      ss = []
      for j in range(_CH):
        ss.append(pltpu.async_copy(yv.at[j], r1_sp.at[idx_d.at[j]], ssem,
                                   add=True))
        ss.append(pltpu.async_copy(dv.at[j], cs_sp.at[idx_s.at[j]], ssem,
                                   add=True))
      for d in ss:
        d.wait()

    plsc.subcore_barrier()
    pltpu.sync_copy(r1_sp.at[pl.ds(s * sl, sl)], slice_v)
    pltpu.sync_copy(slice_v, r1_out.at[pl.ds(w * sl, sl)])
    pltpu.sync_copy(cs_sp.at[pl.ds(s * sl, sl)], slice_v)
    pltpu.sync_copy(slice_v, cs_out.at[pl.ds(w * sl, sl)])

  return k(src2d, dst2d, y_hbm, dinv_hbm)


def _sc_wide_pass(src2d, dst2d, zz, zeros8, npad, rows):
  sl = npad // _NS
  rpw = rows // _NS
  chunks = rpw // _CH
  nrows = _CH * 128

  @functools.partial(
      pl.kernel,
      out_type=[jax.ShapeDtypeStruct((npad, 8), jnp.float32)
                for _ in range(8)],
      mesh=_mesh(),
      compiler_params=pltpu.CompilerParams(use_tc_tiling_on_sc=False),
      scratch_types=[
          pltpu.VMEM((_CH, 128), jnp.int32),
          pltpu.VMEM((_CH, 128), jnp.int32),
          pltpu.VMEM((nrows, 8), jnp.float32),
          pltpu.VMEM((128, 8), jnp.float32),
          pltpu.VMEM_SHARED((npad, 8), jnp.float32),
          pltpu.SemaphoreType.DMA,
          pltpu.SemaphoreType.DMA,
      ],
  )
  def k(src_hbm, dst_hbm, zeros_hbm, *rest):
    zzs = rest[:8]
    outs = rest[8:16]
    idx_s, idx_d, rows_v, zv, acc_sp, gsem, ssem = rest[16:]
    c = lax.axis_index("c")
    s = lax.axis_index("s")
    pltpu.sync_copy(zeros_hbm, zv)

    for g in range(8):
      @pl.when(c == g // 4)
      def _():
        zz_g = zzs[g]
        out_g = outs[g]

        @pl.loop(0, sl // 128)
        def _(i):
          pltpu.sync_copy(zv, acc_sp.at[pl.ds(s * sl + i * 128, 128)])

        plsc.subcore_barrier()

        @pl.loop(0, chunks)
        def _(t):
          base = s * rpw + t * _CH
          pltpu.sync_copy(src_hbm.at[pl.ds(base, _CH)], idx_s)
          pltpu.sync_copy(dst_hbm.at[pl.ds(base, _CH)], idx_d)
          gs = []
          for j in range(_CH):
            gs.append(pltpu.async_copy(zz_g.at[idx_s.at[j]],
                                       rows_v.at[pl.ds(j * 128, 128)], gsem))
          for d in gs:
            d.wait()
          ss = []
          for j in range(_CH):
            ss.append(pltpu.async_copy(rows_v.at[pl.ds(j * 128, 128)],
                                       acc_sp.at[idx_d.at[j]], ssem,
                                       add=True))
          for d in ss:
            d.wait()

        plsc.subcore_barrier()

        @pl.loop(0, sl // 128)
        def _(i):
          pltpu.sync_copy(acc_sp.at[pl.ds(s * sl + i * 128, 128)],
                          rows_v.at[pl.ds(0, 128)])
          pltpu.sync_copy(rows_v.at[pl.ds(0, 128)],
                          out_g.at[pl.ds(s * sl + i * 128, 128)])

  return k(src2d, dst2d, zeros8, *zz)


def _tc_prep(degp, xp2, npad):
  r = npad // 128
  rb = _row_block(r)
  grid = (r // rb,)

  def body(degp_ref, x_ref, dinv_ref, y_ref):
    deg = degp_ref[0] + degp_ref[1] + 1.0
    dinv = lax.rsqrt(deg)
    dinv_ref[...] = dinv
    y_ref[...] = dinv * x_ref[...]

  return pl.pallas_call(
      body,
      grid=grid,
      in_specs=[
          pl.BlockSpec((2, rb, 128), lambda i: (0, i, 0)),
          pl.BlockSpec((rb, 128), lambda i: (i, 0)),
      ],
      out_specs=[
          pl.BlockSpec((rb, 128), lambda i: (i, 0)),
          pl.BlockSpec((rb, 128), lambda i: (i, 0)),
      ],
      out_shape=[
          jax.ShapeDtypeStruct((r, 128), jnp.float32),
          jax.ShapeDtypeStruct((r, 128), jnp.float32),
      ],
  )(degp, xp2)


def _tc_s1(dinv2, y2, r1p, csp, n, npad):
  r = npad // 128
  rb = _row_block(r)
  blk = rb * 128
  grid = (r // rb,)

  def body(dinv_ref, y_ref, r1_ref, cs_ref, s1_ref, cv_ref):
    i = pl.program_id(0)
    dinv = dinv_ref[...]
    r1 = r1_ref[0] + r1_ref[1]
    s1_ref[...] = dinv * (r1 + y_ref[...])
    csum = cs_ref[0] + cs_ref[1]
    cvec = dinv * (csum + dinv)
    row = lax.broadcasted_iota(jnp.int32, (rb, 128), 0)
    col = lax.broadcasted_iota(jnp.int32, (rb, 128), 1)
    gidx = i * blk + row * 128 + col
    cv_ref[...] = jnp.where(gidx < n, cvec, 0.0)

  return pl.pallas_call(
      body,
      grid=grid,
      in_specs=[
          pl.BlockSpec((rb, 128), lambda i: (i, 0)),
          pl.BlockSpec((rb, 128), lambda i: (i, 0)),
          pl.BlockSpec((2, rb, 128), lambda i: (0, i, 0)),
          pl.BlockSpec((2, rb, 128), lambda i: (0, i, 0)),
      ],
      out_specs=[
          pl.BlockSpec((rb, 128), lambda i: (i, 0)),
          pl.BlockSpec((rb, 128), lambda i: (i, 0)),
      ],
      out_shape=[
          jax.ShapeDtypeStruct((r, 128), jnp.float32),
          jax.ShapeDtypeStruct((r, 128), jnp.float32),
      ],
  )(dinv2, y2, r1p, csp)


def _tc_expand(s1n, dinvn, a1, c1, W2, npad):
  blk = 2048
  grid = (npad // blk,)

  def body(s1_ref, dinv_ref, a1_ref, c1_ref, w2_ref, *z_refs):
    h1 = jnp.maximum(s1_ref[...] * a1_ref[...] + c1_ref[...], 0.0)
    z = jnp.dot(h1, w2_ref[...], preferred_element_type=jnp.float32)
    zz = dinv_ref[...] * z
    for g in range(8):
      z_refs[g][...] = zz[:, 8 * g:8 * g + 8]

  return pl.pallas_call(
      body,
      grid=grid,
      in_specs=[
          pl.BlockSpec((blk, 1), lambda i: (i, 0)),
          pl.BlockSpec((blk, 1), lambda i: (i, 0)),
          pl.BlockSpec((1, 64), lambda i: (0, 0)),
          pl.BlockSpec((1, 64), lambda i: (0, 0)),
          pl.BlockSpec((64, 64), lambda i: (0, 0)),
      ],
      out_specs=[pl.BlockSpec((blk, 8), lambda i: (i, 0))] * 8,
      out_shape=[jax.ShapeDtypeStruct((npad, 8), jnp.float32)
                 for _ in range(8)],
  )(s1n, dinvn, a1, c1, W2)


def _tc_final(r2g, zzg, dinvn, cvn, gs2, cb2, W3, a3, cc3,
              Wl1, bl1, Wl2, bl2, npad):
  blk = 2048
  grid = (npad // blk,)
  last = npad // blk - 1

  def body(*refs):
    r_refs = refs[:8]
    z_refs = refs[8:16]
    (dinv_ref, cv_ref, gs2_ref, cb2_ref, w3_ref, a3_ref, cc3_ref,
     wl1_ref, bl1_ref, wl2_ref, bl2_ref, out_ref, acc_ref) = refs[16:]
    i = pl.program_id(0)
    rr = jnp.concatenate([r[...] for r in r_refs], axis=1)
    zz = jnp.concatenate([z[...] for z in z_refs], axis=1)
    out2 = dinv_ref[...] * (rr + zz)
    h2 = jnp.maximum(out2 * gs2_ref[...] + cb2_ref[...], 0.0)
    pv = lax.dot_general(cv_ref[...], h2, (((0,), (0,)), ((), ())),
                         preferred_element_type=jnp.float32)

    @pl.when(i == 0)
    def _():
      acc_ref[...] = jnp.zeros_like(acc_ref)

    acc_ref[...] += pv

    @pl.when(i == last)
    def _():
      v = acc_ref[...]
      m = jnp.dot(v, w3_ref[...], preferred_element_type=jnp.float32)
      m = m * a3_ref[...] + cc3_ref[...]
      h = jnp.maximum(
          jnp.dot(m, wl1_ref[...], preferred_element_type=jnp.float32)
          + bl1_ref[...], 0.0)
      logits = jnp.dot(h, wl2_ref[...],
                       preferred_element_type=jnp.float32) + bl2_ref[...]
      mx = jnp.max(logits, axis=1, keepdims=True)
      ex = jnp.exp(logits - mx)
      out_ref[...] = ex / jnp.sum(ex, axis=1, keepdims=True)

  return pl.pallas_call(
      body,
      grid=grid,
      in_specs=[pl.BlockSpec((blk, 8), lambda i: (i, 0))] * 16
      + [
          pl.BlockSpec((blk, 1), lambda i: (i, 0)),
          pl.BlockSpec((blk, 1), lambda i: (i, 0)),
          pl.BlockSpec((1, 64), lambda i: (0, 0)),
          pl.BlockSpec((1, 64), lambda i: (0, 0)),
          pl.BlockSpec((64, 64), lambda i: (0, 0)),
          pl.BlockSpec((1, 64), lambda i: (0, 0)),
          pl.BlockSpec((1, 64), lambda i: (0, 0)),
          pl.BlockSpec((64, 32), lambda i: (0, 0)),
          pl.BlockSpec((1, 32), lambda i: (0, 0)),
          pl.BlockSpec((32, 3), lambda i: (0, 0)),
          pl.BlockSpec((1, 3), lambda i: (0, 0)),
      ],
      out_specs=pl.BlockSpec((1, 3), lambda i: (0, 0)),
      out_shape=jax.ShapeDtypeStruct((1, 3), jnp.float32),
      scratch_shapes=[pltpu.VMEM((1, 64), jnp.float32)],
  )(*r2g, *zzg, dinvn, cvn, gs2, cb2, W3, a3, cc3, Wl1, bl1, Wl2, bl2)


def kernel(x, edge_index, W1, b1, g1, be1, W2, b2, g2, be2, W3, b3, g3, be3,
           Wl1, bl1, Wl2, bl2):
  n = x.shape[0]
  e = edge_index.shape[1]
  npad, rows = _pad_sizes(n, e)
  epad = rows * 128

  xp = jnp.zeros((npad,), jnp.float32).at[:n].set(x[:, 0])
  pad_idx = (n + (jnp.arange(epad - e, dtype=jnp.int32) % 256))
  src = jnp.concatenate([edge_index[0].astype(jnp.int32), pad_idx])
  dst = jnp.concatenate([edge_index[1].astype(jnp.int32), pad_idx])
  src2d = src.reshape(rows, 128)
  dst2d = dst.reshape(rows, 128)
  xp2 = xp.reshape(npad // 128, 128)

  gs1 = g1 / jnp.sqrt(1.0 + _EPS)
  gs2 = g2 / jnp.sqrt(1.0 + _EPS)
  gs3 = g3 / jnp.sqrt(1.0 + _EPS)
  a1 = (W1[0] * gs1).reshape(1, 64)
  c1 = (b1 * gs1 + be1).reshape(1, 64)
  cb2 = (b2 * gs2 + be2).reshape(1, 64)
  a3 = (gs3 / np.float32(n)).reshape(1, 64)
  cc3 = (b3 * gs3 + be3).reshape(1, 64)

  degp = _sc_degree(dst2d, npad, rows)
  degp = degp.reshape(_NC, npad // 128, 128)
  dinv2, y2 = _tc_prep(degp, xp2, npad)

  r1p, csp = _sc_scalar_pass(src2d, dst2d, y2.reshape(npad),
                             dinv2.reshape(npad), npad, rows)
  r1p = r1p.reshape(_NC, npad // 128, 128)
  csp = csp.reshape(_NC, npad // 128, 128)
  s1_2d, cv2 = _tc_s1(dinv2, y2, r1p, csp, n, npad)
  dinvn = dinv2.reshape(npad, 1)
  zzg = _tc_expand(s1_2d.reshape(npad, 1), dinvn, a1, c1, W2, npad)

  zeros8 = jnp.zeros((128, 8), jnp.float32)
  r2g = _sc_wide_pass(src2d, dst2d, zzg, zeros8, npad, rows)
  out = _tc_final(r2g, zzg, dinvn, cv2.reshape(npad, 1),
                  gs2.reshape(1, 64), cb2, W3, a3, cc3, Wl1,
                  bl1.reshape(1, 32), Wl2.reshape(32, 3),
                  bl2.reshape(1, 3), npad)
  return out


# speedup vs baseline: 24.5646x; 1.1225x over previous
import functools

import jax
import jax.numpy as jnp
import numpy as np
from jax import lax
from jax.experimental import pallas as pl
from jax.experimental.pallas import tpu as pltpu
from jax.experimental.pallas import tpu_sc as plsc

_EPS = 1e-5

_NC = 2
_NS = 16
_NW = _NC * _NS
_CH = 8


def _pad_sizes(n, e):
  npad = ((n + 256 + 2047) // 2048) * 2048
  rows = -(-e // 128)
  rows = -(-rows // (_NW * _CH)) * (_NW * _CH)
  return npad, rows


def _row_block(r):
  return 56 if r % 56 == 0 else 16


def _mesh():
  return plsc.VectorSubcoreMesh(core_axis_name="c", subcore_axis_name="s",
                                num_cores=_NC, num_subcores=_NS)


def _sc_degree(dst2d, npad, rows):
  sl = npad // _NS
  rpw = rows // _NW
  chunks = rpw // _CH

  @functools.partial(
      pl.kernel,
      out_type=jax.ShapeDtypeStruct((_NW * sl,), jnp.float32),
      mesh=_mesh(),
      scratch_types=[
          pltpu.VMEM((_CH, 128), jnp.int32),
          pltpu.VMEM((128,), jnp.float32),
          pltpu.VMEM((sl,), jnp.float32),
          pltpu.VMEM_SHARED((npad,), jnp.float32),
          pltpu.SemaphoreType.DMA,
      ],
  )
  def k(dst_hbm, out_hbm, idx_v, ones_v, slice_v, acc_sp, sem):
    c = lax.axis_index("c")
    s = lax.axis_index("s")
    w = c * _NS + s

    @pl.loop(0, 8)
    def _(i):
      ones_v[pl.ds(i * 16, 16)] = jnp.ones((16,), jnp.float32)

    @pl.loop(0, sl // 16)
    def _(i):
      slice_v[pl.ds(i * 16, 16)] = jnp.zeros((16,), jnp.float32)

    pltpu.sync_copy(slice_v, acc_sp.at[pl.ds(s * sl, sl)])
    plsc.subcore_barrier()

    @pl.loop(0, chunks)
    def _(t):
      base = w * rpw + t * _CH
      pltpu.sync_copy(dst_hbm.at[pl.ds(base, _CH)], idx_v)
      ds = []
      for j in range(_CH):
        ds.append(pltpu.async_copy(ones_v, acc_sp.at[idx_v.at[j]], sem,
                                   add=True))
      for d in ds:
        d.wait()

    plsc.subcore_barrier()
    pltpu.sync_copy(acc_sp.at[pl.ds(s * sl, sl)], slice_v)
    pltpu.sync_copy(slice_v, out_hbm.at[pl.ds(w * sl, sl)])

  return k(dst2d)


def _sc_scalar_pass(src2d, dst2d, y_hbm, dinv_hbm, npad, rows):
  sl = npad // _NS
  rpw = rows // _NW
  chunks = rpw // _CH

  @functools.partial(
      pl.kernel,
      out_type=[
          jax.ShapeDtypeStruct((_NW * sl,), jnp.float32),
          jax.ShapeDtypeStruct((_NW * sl,), jnp.float32),
      ],
      mesh=_mesh(),
      scratch_types=[
          pltpu.VMEM((_CH, 128), jnp.int32),
          pltpu.VMEM((_CH, 128), jnp.int32),
          pltpu.VMEM((_CH, 128), jnp.float32),
          pltpu.VMEM((_CH, 128), jnp.float32),
          pltpu.VMEM((sl,), jnp.float32),
          pltpu.VMEM_SHARED((npad,), jnp.float32),
          pltpu.VMEM_SHARED((npad,), jnp.float32),
          pltpu.VMEM_SHARED((npad,), jnp.float32),
          pltpu.VMEM_SHARED((npad,), jnp.float32),
          pltpu.SemaphoreType.DMA,
          pltpu.SemaphoreType.DMA,
      ],
  )
  def k(src_hbm, dst_hbm, y_in, dinv_in, r1_out, cs_out,
        idx_s, idx_d, yv, dv, slice_v, y_sp, dinv_sp, r1_sp, cs_sp,
        gsem, ssem):
    c = lax.axis_index("c")
    s = lax.axis_index("s")
    w = c * _NS + s

    pltpu.sync_copy(y_in.at[pl.ds(s * sl, sl)], slice_v)
    pltpu.sync_copy(slice_v, y_sp.at[pl.ds(s * sl, sl)])
    pltpu.sync_copy(dinv_in.at[pl.ds(s * sl, sl)], slice_v)
    pltpu.sync_copy(slice_v, dinv_sp.at[pl.ds(s * sl, sl)])

    @pl.loop(0, sl // 16)
    def _(i):
      slice_v[pl.ds(i * 16, 16)] = jnp.zeros((16,), jnp.float32)

    pltpu.sync_copy(slice_v, r1_sp.at[pl.ds(s * sl, sl)])
    pltpu.sync_copy(slice_v, cs_sp.at[pl.ds(s * sl, sl)])
    plsc.subcore_barrier()

    @pl.loop(0, chunks)
    def _(t):
      base = w * rpw + t * _CH
      pltpu.sync_copy(src_hbm.at[pl.ds(base, _CH)], idx_s)
      pltpu.sync_copy(dst_hbm.at[pl.ds(base, _CH)], idx_d)
      gs = []
      for j in range(_CH):
        gs.append(pltpu.async_copy(y_sp.at[idx_s.at[j]], yv.at[j], gsem))
        gs.append(pltpu.async_copy(dinv_sp.at[idx_d.at[j]], dv.at[j], gsem))
      for d in gs:
        d.wait()
      ss = []
      for j in range(_CH):
        ss.append(pltpu.async_copy(yv.at[j], r1_sp.at[idx_d.at[j]], ssem,
                                   add=True))
        ss.append(pltpu.async_copy(dv.at[j], cs_sp.at[idx_s.at[j]], ssem,
                                   add=True))
      for d in ss:
        d.wait()

    plsc.subcore_barrier()
    pltpu.sync_copy(r1_sp.at[pl.ds(s * sl, sl)], slice_v)
    pltpu.sync_copy(slice_v, r1_out.at[pl.ds(w * sl, sl)])
    pltpu.sync_copy(cs_sp.at[pl.ds(s * sl, sl)], slice_v)
    pltpu.sync_copy(slice_v, cs_out.at[pl.ds(w * sl, sl)])

  return k(src2d, dst2d, y_hbm, dinv_hbm)


def _sc_wide_pass(src2d, dst2d, zz, zeros8, npad, rows):
  sl = npad // _NS
  rpw = rows // _NS
  chunks = rpw // _CH
  assert chunks % 2 == 0
  nrows = _CH * 128

  @functools.partial(
      pl.kernel,
      out_type=[jax.ShapeDtypeStruct((npad, 8), jnp.float32)
                for _ in range(8)],
      mesh=_mesh(),
      compiler_params=pltpu.CompilerParams(use_tc_tiling_on_sc=False),
      scratch_types=[
          pltpu.VMEM((_CH, 128), jnp.int32),
          pltpu.VMEM((_CH, 128), jnp.int32),
          pltpu.VMEM((_CH, 128), jnp.int32),
          pltpu.VMEM((_CH, 128), jnp.int32),
          pltpu.VMEM((nrows, 8), jnp.float32),
          pltpu.VMEM((nrows, 8), jnp.float32),
          pltpu.VMEM((128, 8), jnp.float32),
          pltpu.VMEM_SHARED((npad, 8), jnp.float32),
          pltpu.SemaphoreType.DMA,
          pltpu.SemaphoreType.DMA,
          pltpu.SemaphoreType.DMA,
          pltpu.SemaphoreType.DMA,
      ],
  )
  def k(src_hbm, dst_hbm, zeros_hbm, *rest):
    zzs = rest[:8]
    outs = rest[8:16]
    (idx_s0, idx_s1, idx_d0, idx_d1, rows_v0, rows_v1, zv, acc_sp,
     gsem0, gsem1, ssem0, ssem1) = rest[16:]
    idx_s = [idx_s0, idx_s1]
    idx_d = [idx_d0, idx_d1]
    rows_v = [rows_v0, rows_v1]
    gsem = [gsem0, gsem1]
    ssem = [ssem0, ssem1]
    c = lax.axis_index("c")
    s = lax.axis_index("s")
    pltpu.sync_copy(zeros_hbm, zv)

    for g in range(8):
      @pl.when(c == g // 4)
      def _():
        zz_g = zzs[g]
        out_g = outs[g]

        def stage(q, b):
          pltpu.sync_copy(src_hbm.at[pl.ds(s * rpw + q * _CH, _CH)],
                          idx_s[b])
          pltpu.sync_copy(dst_hbm.at[pl.ds(s * rpw + q * _CH, _CH)],
                          idx_d[b])

        def fire_gathers(b):
          for j in range(_CH):
            pltpu.async_copy(zz_g.at[idx_s[b].at[j]],
                             rows_v[b].at[pl.ds(j * 128, 128)], gsem[b])

        def drain_gathers(b):
          for j in range(_CH):
            pltpu.make_async_copy(zz_g.at[idx_s[b].at[j]],
                                  rows_v[b].at[pl.ds(j * 128, 128)],
                                  gsem[b]).wait()

        def fire_scatters(b):
          for j in range(_CH):
            pltpu.async_copy(rows_v[b].at[pl.ds(j * 128, 128)],
                             acc_sp.at[idx_d[b].at[j]], ssem[b], add=True)

        def drain_scatters(b):
          for j in range(_CH):
            pltpu.make_async_copy(rows_v[b].at[pl.ds(j * 128, 128)],
                                  acc_sp.at[idx_d[b].at[j]],
                                  ssem[b]).wait()

        @pl.loop(0, sl // 128)
        def _(i):
          pltpu.sync_copy(zv, acc_sp.at[pl.ds(s * sl + i * 128, 128)])

        plsc.subcore_barrier()

        stage(0, 0)
        fire_gathers(0)

        @pl.loop(0, chunks, step=2)
        def _(t):
          for b in range(2):
            q = t + b

            @pl.when(q >= 1)
            def _():
              drain_scatters(1 - b)

            @pl.when(q + 1 < chunks)
            def _():
              stage(q + 1, 1 - b)
              fire_gathers(1 - b)

            drain_gathers(b)
            fire_scatters(b)

        drain_scatters(1)
        plsc.subcore_barrier()

        @pl.loop(0, sl // 128)
        def _(i):
          pltpu.sync_copy(acc_sp.at[pl.ds(s * sl + i * 128, 128)],
                          rows_v0.at[pl.ds(0, 128)])
          pltpu.sync_copy(rows_v0.at[pl.ds(0, 128)],
                          out_g.at[pl.ds(s * sl + i * 128, 128)])

  return k(src2d, dst2d, zeros8, *zz)


def _tc_prep(degp, xp2, npad):
  r = npad // 128
  rb = _row_block(r)
  grid = (r // rb,)

  def body(degp_ref, x_ref, dinv_ref, y_ref):
    deg = degp_ref[0] + degp_ref[1] + 1.0
    dinv = lax.rsqrt(deg)
    dinv_ref[...] = dinv
    y_ref[...] = dinv * x_ref[...]

  return pl.pallas_call(
      body,
      grid=grid,
      in_specs=[
          pl.BlockSpec((2, rb, 128), lambda i: (0, i, 0)),
          pl.BlockSpec((rb, 128), lambda i: (i, 0)),
      ],
      out_specs=[
          pl.BlockSpec((rb, 128), lambda i: (i, 0)),
          pl.BlockSpec((rb, 128), lambda i: (i, 0)),
      ],
      out_shape=[
          jax.ShapeDtypeStruct((r, 128), jnp.float32),
          jax.ShapeDtypeStruct((r, 128), jnp.float32),
      ],
  )(degp, xp2)


def _tc_s1(dinv2, y2, r1p, csp, n, npad):
  r = npad // 128
  rb = _row_block(r)
  blk = rb * 128
  grid = (r // rb,)

  def body(dinv_ref, y_ref, r1_ref, cs_ref, s1_ref, cv_ref):
    i = pl.program_id(0)
    dinv = dinv_ref[...]
    r1 = r1_ref[0] + r1_ref[1]
    s1_ref[...] = dinv * (r1 + y_ref[...])
    csum = cs_ref[0] + cs_ref[1]
    cvec = dinv * (csum + dinv)
    row = lax.broadcasted_iota(jnp.int32, (rb, 128), 0)
    col = lax.broadcasted_iota(jnp.int32, (rb, 128), 1)
    gidx = i * blk + row * 128 + col
    cv_ref[...] = jnp.where(gidx < n, cvec, 0.0)

  return pl.pallas_call(
      body,
      grid=grid,
      in_specs=[
          pl.BlockSpec((rb, 128), lambda i: (i, 0)),
          pl.BlockSpec((rb, 128), lambda i: (i, 0)),
          pl.BlockSpec((2, rb, 128), lambda i: (0, i, 0)),
          pl.BlockSpec((2, rb, 128), lambda i: (0, i, 0)),
      ],
      out_specs=[
          pl.BlockSpec((rb, 128), lambda i: (i, 0)),
          pl.BlockSpec((rb, 128), lambda i: (i, 0)),
      ],
      out_shape=[
          jax.ShapeDtypeStruct((r, 128), jnp.float32),
          jax.ShapeDtypeStruct((r, 128), jnp.float32),
      ],
  )(dinv2, y2, r1p, csp)


def _tc_expand(s1n, dinvn, a1, c1, W2, npad):
  blk = 2048
  grid = (npad // blk,)

  def body(s1_ref, dinv_ref, a1_ref, c1_ref, w2_ref, *z_refs):
    h1 = jnp.maximum(s1_ref[...] * a1_ref[...] + c1_ref[...], 0.0)
    z = jnp.dot(h1, w2_ref[...], preferred_element_type=jnp.float32)
    zz = dinv_ref[...] * z
    for g in range(8):
      z_refs[g][...] = zz[:, 8 * g:8 * g + 8]

  return pl.pallas_call(
      body,
      grid=grid,
      in_specs=[
          pl.BlockSpec((blk, 1), lambda i: (i, 0)),
          pl.BlockSpec((blk, 1), lambda i: (i, 0)),
          pl.BlockSpec((1, 64), lambda i: (0, 0)),
          pl.BlockSpec((1, 64), lambda i: (0, 0)),
          pl.BlockSpec((64, 64), lambda i: (0, 0)),
      ],
      out_specs=[pl.BlockSpec((blk, 8), lambda i: (i, 0))] * 8,
      out_shape=[jax.ShapeDtypeStruct((npad, 8), jnp.float32)
                 for _ in range(8)],
  )(s1n, dinvn, a1, c1, W2)


def _tc_final(r2g, zzg, dinvn, cvn, gs2, cb2, W3, a3, cc3,
              Wl1, bl1, Wl2, bl2, npad):
  blk = 2048
  grid = (npad // blk,)
  last = npad // blk - 1

  def body(*refs):
    r_refs = refs[:8]
    z_refs = refs[8:16]
    (dinv_ref, cv_ref, gs2_ref, cb2_ref, w3_ref, a3_ref, cc3_ref,
     wl1_ref, bl1_ref, wl2_ref, bl2_ref, out_ref, acc_ref) = refs[16:]
    i = pl.program_id(0)
    rr = jnp.concatenate([r[...] for r in r_refs], axis=1)
    zz = jnp.concatenate([z[...] for z in z_refs], axis=1)
    out2 = dinv_ref[...] * (rr + zz)
    h2 = jnp.maximum(out2 * gs2_ref[...] + cb2_ref[...], 0.0)
    pv = lax.dot_general(cv_ref[...], h2, (((0,), (0,)), ((), ())),
                         preferred_element_type=jnp.float32)

    @pl.when(i == 0)
    def _():
      acc_ref[...] = jnp.zeros_like(acc_ref)

    acc_ref[...] += pv

    @pl.when(i == last)
    def _():
      v = acc_ref[...]
      m = jnp.dot(v, w3_ref[...], preferred_element_type=jnp.float32)
      m = m * a3_ref[...] + cc3_ref[...]
      h = jnp.maximum(
          jnp.dot(m, wl1_ref[...], preferred_element_type=jnp.float32)
          + bl1_ref[...], 0.0)
      logits = jnp.dot(h, wl2_ref[...],
                       preferred_element_type=jnp.float32) + bl2_ref[...]
      mx = jnp.max(logits, axis=1, keepdims=True)
      ex = jnp.exp(logits - mx)
      out_ref[...] = ex / jnp.sum(ex, axis=1, keepdims=True)

  return pl.pallas_call(
      body,
      grid=grid,
      in_specs=[pl.BlockSpec((blk, 8), lambda i: (i, 0))] * 16
      + [
          pl.BlockSpec((blk, 1), lambda i: (i, 0)),
          pl.BlockSpec((blk, 1), lambda i: (i, 0)),
          pl.BlockSpec((1, 64), lambda i: (0, 0)),
          pl.BlockSpec((1, 64), lambda i: (0, 0)),
          pl.BlockSpec((64, 64), lambda i: (0, 0)),
          pl.BlockSpec((1, 64), lambda i: (0, 0)),
          pl.BlockSpec((1, 64), lambda i: (0, 0)),
          pl.BlockSpec((64, 32), lambda i: (0, 0)),
          pl.BlockSpec((1, 32), lambda i: (0, 0)),
          pl.BlockSpec((32, 3), lambda i: (0, 0)),
          pl.BlockSpec((1, 3), lambda i: (0, 0)),
      ],
      out_specs=pl.BlockSpec((1, 3), lambda i: (0, 0)),
      out_shape=jax.ShapeDtypeStruct((1, 3), jnp.float32),
      scratch_shapes=[pltpu.VMEM((1, 64), jnp.float32)],
  )(*r2g, *zzg, dinvn, cvn, gs2, cb2, W3, a3, cc3, Wl1, bl1, Wl2, bl2)


def kernel(x, edge_index, W1, b1, g1, be1, W2, b2, g2, be2, W3, b3, g3, be3,
           Wl1, bl1, Wl2, bl2):
  n = x.shape[0]
  e = edge_index.shape[1]
  npad, rows = _pad_sizes(n, e)
  epad = rows * 128

  xp = jnp.zeros((npad,), jnp.float32).at[:n].set(x[:, 0])
  pad_idx = (n + (jnp.arange(epad - e, dtype=jnp.int32) % 256))
  src = jnp.concatenate([edge_index[0].astype(jnp.int32), pad_idx])
  dst = jnp.concatenate([edge_index[1].astype(jnp.int32), pad_idx])
  src2d = src.reshape(rows, 128)
  dst2d = dst.reshape(rows, 128)
  xp2 = xp.reshape(npad // 128, 128)

  gs1 = g1 / jnp.sqrt(1.0 + _EPS)
  gs2 = g2 / jnp.sqrt(1.0 + _EPS)
  gs3 = g3 / jnp.sqrt(1.0 + _EPS)
  a1 = (W1[0] * gs1).reshape(1, 64)
  c1 = (b1 * gs1 + be1).reshape(1, 64)
  cb2 = (b2 * gs2 + be2).reshape(1, 64)
  a3 = (gs3 / np.float32(n)).reshape(1, 64)
  cc3 = (b3 * gs3 + be3).reshape(1, 64)

  degp = _sc_degree(dst2d, npad, rows)
  degp = degp.reshape(_NC, npad // 128, 128)
  dinv2, y2 = _tc_prep(degp, xp2, npad)

  r1p, csp = _sc_scalar_pass(src2d, dst2d, y2.reshape(npad),
                             dinv2.reshape(npad), npad, rows)
  r1p = r1p.reshape(_NC, npad // 128, 128)
  csp = csp.reshape(_NC, npad // 128, 128)
  s1_2d, cv2 = _tc_s1(dinv2, y2, r1p, csp, n, npad)
  dinvn = dinv2.reshape(npad, 1)
  zzg = _tc_expand(s1_2d.reshape(npad, 1), dinvn, a1, c1, W2, npad)

  zeros8 = jnp.zeros((128, 8), jnp.float32)
  r2g = _sc_wide_pass(src2d, dst2d, zzg, zeros8, npad, rows)
  out = _tc_final(r2g, zzg, dinvn, cv2.reshape(npad, 1),
                  gs2.reshape(1, 64), cb2, W3, a3, cc3, Wl1,
                  bl1.reshape(1, 32), Wl2.reshape(32, 3),
                  bl2.reshape(1, 3), npad)
  return out


# speedup vs baseline: 26.6883x; 1.0865x over previous
import functools

import jax
import jax.numpy as jnp
import numpy as np
from jax import lax
from jax.experimental import pallas as pl
from jax.experimental.pallas import tpu as pltpu
from jax.experimental.pallas import tpu_sc as plsc

_EPS = 1e-5

_NC = 2
_NS = 16
_NW = _NC * _NS
_CH = 16


def _pad_sizes(n, e):
  npad = ((n + 256 + 2047) // 2048) * 2048
  rows = -(-e // 128)
  rows = -(-rows // (_NW * _CH)) * (_NW * _CH)
  return npad, rows


def _row_block(r):
  return 56 if r % 56 == 0 else 16


def _mesh():
  return plsc.VectorSubcoreMesh(core_axis_name="c", subcore_axis_name="s",
                                num_cores=_NC, num_subcores=_NS)


def _sc_degree(dst2d, npad, rows):
  sl = npad // _NS
  rpw = rows // _NW
  chunks = rpw // _CH

  @functools.partial(
      pl.kernel,
      out_type=jax.ShapeDtypeStruct((_NW * sl,), jnp.float32),
      mesh=_mesh(),
      scratch_types=[
          pltpu.VMEM((_CH, 128), jnp.int32),
          pltpu.VMEM((128,), jnp.float32),
          pltpu.VMEM((sl,), jnp.float32),
          pltpu.VMEM_SHARED((npad,), jnp.float32),
          pltpu.SemaphoreType.DMA,
      ],
  )
  def k(dst_hbm, out_hbm, idx_v, ones_v, slice_v, acc_sp, sem):
    c = lax.axis_index("c")
    s = lax.axis_index("s")
    w = c * _NS + s

    @pl.loop(0, 8)
    def _(i):
      ones_v[pl.ds(i * 16, 16)] = jnp.ones((16,), jnp.float32)

    @pl.loop(0, sl // 16)
    def _(i):
      slice_v[pl.ds(i * 16, 16)] = jnp.zeros((16,), jnp.float32)

    pltpu.sync_copy(slice_v, acc_sp.at[pl.ds(s * sl, sl)])
    plsc.subcore_barrier()

    @pl.loop(0, chunks)
    def _(t):
      base = w * rpw + t * _CH
      pltpu.sync_copy(dst_hbm.at[pl.ds(base, _CH)], idx_v)
      for j in range(_CH):
        pltpu.async_copy(ones_v, acc_sp.at[idx_v.at[j]], sem, add=True)
      pltpu.make_async_copy(dst_hbm.at[pl.ds(0, _CH)], idx_v, sem).wait()

    plsc.subcore_barrier()
    pltpu.sync_copy(acc_sp.at[pl.ds(s * sl, sl)], slice_v)
    pltpu.sync_copy(slice_v, out_hbm.at[pl.ds(w * sl, sl)])

  return k(dst2d)


def _sc_scalar_pass(src2d, dst2d, y_hbm, dinv_hbm, npad, rows):
  sl = npad // _NS
  rpw = rows // _NW
  chunks = rpw // _CH

  @functools.partial(
      pl.kernel,
      out_type=[
          jax.ShapeDtypeStruct((_NW * sl,), jnp.float32),
          jax.ShapeDtypeStruct((_NW * sl,), jnp.float32),
      ],
      mesh=_mesh(),
      scratch_types=[
          pltpu.VMEM((_CH, 128), jnp.int32),
          pltpu.VMEM((_CH, 128), jnp.int32),
          pltpu.VMEM((_CH, 128), jnp.float32),
          pltpu.VMEM((_CH, 128), jnp.float32),
          pltpu.VMEM((sl,), jnp.float32),
          pltpu.VMEM_SHARED((npad,), jnp.float32),
          pltpu.VMEM_SHARED((npad,), jnp.float32),
          pltpu.VMEM_SHARED((npad,), jnp.float32),
          pltpu.VMEM_SHARED((npad,), jnp.float32),
          pltpu.SemaphoreType.DMA,
          pltpu.SemaphoreType.DMA,
      ],
  )
  def k(src_hbm, dst_hbm, y_in, dinv_in, r1_out, cs_out,
        idx_s, idx_d, yv, dv, slice_v, y_sp, dinv_sp, r1_sp, cs_sp,
        gsem, ssem):
    c = lax.axis_index("c")
    s = lax.axis_index("s")
    w = c * _NS + s

    pltpu.sync_copy(y_in.at[pl.ds(s * sl, sl)], slice_v)
    pltpu.sync_copy(slice_v, y_sp.at[pl.ds(s * sl, sl)])
    pltpu.sync_copy(dinv_in.at[pl.ds(s * sl, sl)], slice_v)
    pltpu.sync_copy(slice_v, dinv_sp.at[pl.ds(s * sl, sl)])

    @pl.loop(0, sl // 16)
    def _(i):
      slice_v[pl.ds(i * 16, 16)] = jnp.zeros((16,), jnp.float32)

    pltpu.sync_copy(slice_v, r1_sp.at[pl.ds(s * sl, sl)])
    pltpu.sync_copy(slice_v, cs_sp.at[pl.ds(s * sl, sl)])
    plsc.subcore_barrier()

    @pl.loop(0, chunks)
    def _(t):
      base = w * rpw + t * _CH
      pltpu.sync_copy(src_hbm.at[pl.ds(base, _CH)], idx_s)
      pltpu.sync_copy(dst_hbm.at[pl.ds(base, _CH)], idx_d)
      for j in range(_CH):
        pltpu.async_copy(y_sp.at[idx_s.at[j]], yv.at[j], gsem)
        pltpu.async_copy(dinv_sp.at[idx_d.at[j]], dv.at[j], gsem)
      pltpu.make_async_copy(src_hbm.at[pl.ds(0, _CH)], yv, gsem).wait()
      pltpu.make_async_copy(src_hbm.at[pl.ds(0, _CH)], dv, gsem).wait()
      for j in range(_CH):
        pltpu.async_copy(yv.at[j], r1_sp.at[idx_d.at[j]], ssem, add=True)
        pltpu.async_copy(dv.at[j], cs_sp.at[idx_s.at[j]], ssem, add=True)
      pltpu.make_async_copy(src_hbm.at[pl.ds(0, _CH)], yv, ssem).wait()
      pltpu.make_async_copy(src_hbm.at[pl.ds(0, _CH)], dv, ssem).wait()

    plsc.subcore_barrier()
    pltpu.sync_copy(r1_sp.at[pl.ds(s * sl, sl)], slice_v)
    pltpu.sync_copy(slice_v, r1_out.at[pl.ds(w * sl, sl)])
    pltpu.sync_copy(cs_sp.at[pl.ds(s * sl, sl)], slice_v)
    pltpu.sync_copy(slice_v, cs_out.at[pl.ds(w * sl, sl)])

  return k(src2d, dst2d, y_hbm, dinv_hbm)


def _sc_wide_pass(src2d, dst2d, zz, zeros8, npad, rows):
  sl = npad // _NS
  rpw = rows // _NS
  chunks = rpw // _CH
  assert chunks % 2 == 0
  nrows = _CH * 128

  @functools.partial(
      pl.kernel,
      out_type=[jax.ShapeDtypeStruct((npad, 8), jnp.float32)
                for _ in range(8)],
      mesh=_mesh(),
      compiler_params=pltpu.CompilerParams(use_tc_tiling_on_sc=False),
      scratch_types=[
          pltpu.VMEM((_CH, 128), jnp.int32),
          pltpu.VMEM((_CH, 128), jnp.int32),
          pltpu.VMEM((_CH, 128), jnp.int32),
          pltpu.VMEM((_CH, 128), jnp.int32),
          pltpu.VMEM((nrows, 8), jnp.float32),
          pltpu.VMEM((nrows, 8), jnp.float32),
          pltpu.VMEM((128, 8), jnp.float32),
          pltpu.VMEM_SHARED((npad, 8), jnp.float32),
          pltpu.SemaphoreType.DMA,
          pltpu.SemaphoreType.DMA,
          pltpu.SemaphoreType.DMA,
          pltpu.SemaphoreType.DMA,
      ],
  )
  def k(src_hbm, dst_hbm, zeros_hbm, *rest):
    zzs = rest[:8]
    outs = rest[8:16]
    (idx_s0, idx_s1, idx_d0, idx_d1, rows_v0, rows_v1, zv, acc_sp,
     gsem0, gsem1, ssem0, ssem1) = rest[16:]
    idx_s = [idx_s0, idx_s1]
    idx_d = [idx_d0, idx_d1]
    rows_v = [rows_v0, rows_v1]
    gsem = [gsem0, gsem1]
    ssem = [ssem0, ssem1]
    c = lax.axis_index("c")
    s = lax.axis_index("s")
    pltpu.sync_copy(zeros_hbm, zv)

    for g in range(8):
      @pl.when(c == g // 4)
      def _():
        zz_g = zzs[g]
        out_g = outs[g]

        def stage(q, b):
          pltpu.sync_copy(src_hbm.at[pl.ds(s * rpw + q * _CH, _CH)],
                          idx_s[b])
          pltpu.sync_copy(dst_hbm.at[pl.ds(s * rpw + q * _CH, _CH)],
                          idx_d[b])

        def fire_gathers(b):
          for j in range(_CH):
            pltpu.async_copy(zz_g.at[idx_s[b].at[j]],
                             rows_v[b].at[pl.ds(j * 128, 128)], gsem[b])

        def drain_gathers(b):
          pltpu.make_async_copy(zz_g.at[pl.ds(0, nrows)], rows_v[b],
                                gsem[b]).wait()

        def fire_scatters(b):
          for j in range(_CH):
            pltpu.async_copy(rows_v[b].at[pl.ds(j * 128, 128)],
                             acc_sp.at[idx_d[b].at[j]], ssem[b], add=True)

        def drain_scatters(b):
          pltpu.make_async_copy(zz_g.at[pl.ds(0, nrows)], rows_v[b],
                                ssem[b]).wait()

        @pl.loop(0, sl // 128)
        def _(i):
          pltpu.sync_copy(zv, acc_sp.at[pl.ds(s * sl + i * 128, 128)])

        plsc.subcore_barrier()

        stage(0, 0)
        fire_gathers(0)

        @pl.loop(0, chunks, step=2)
        def _(t):
          for b in range(2):
            q = t + b

            @pl.when(q >= 1)
            def _():
              drain_scatters(1 - b)

            @pl.when(q + 1 < chunks)
            def _():
              stage(q + 1, 1 - b)
              fire_gathers(1 - b)

            drain_gathers(b)
            fire_scatters(b)

        drain_scatters(1)
        plsc.subcore_barrier()

        @pl.loop(0, sl // 128)
        def _(i):
          pltpu.sync_copy(acc_sp.at[pl.ds(s * sl + i * 128, 128)],
                          rows_v0.at[pl.ds(0, 128)])
          pltpu.sync_copy(rows_v0.at[pl.ds(0, 128)],
                          out_g.at[pl.ds(s * sl + i * 128, 128)])

  return k(src2d, dst2d, zeros8, *zz)


def _tc_prep(degp, xp2, npad):
  r = npad // 128
  rb = _row_block(r)
  grid = (r // rb,)

  def body(degp_ref, x_ref, dinv_ref, y_ref):
    deg = degp_ref[0] + degp_ref[1] + 1.0
    dinv = lax.rsqrt(deg)
    dinv_ref[...] = dinv
    y_ref[...] = dinv * x_ref[...]

  return pl.pallas_call(
      body,
      grid=grid,
      in_specs=[
          pl.BlockSpec((2, rb, 128), lambda i: (0, i, 0)),
          pl.BlockSpec((rb, 128), lambda i: (i, 0)),
      ],
      out_specs=[
          pl.BlockSpec((rb, 128), lambda i: (i, 0)),
          pl.BlockSpec((rb, 128), lambda i: (i, 0)),
      ],
      out_shape=[
          jax.ShapeDtypeStruct((r, 128), jnp.float32),
          jax.ShapeDtypeStruct((r, 128), jnp.float32),
      ],
  )(degp, xp2)


def _tc_s1(dinv2, y2, r1p, csp, n, npad):
  r = npad // 128
  rb = _row_block(r)
  blk = rb * 128
  grid = (r // rb,)

  def body(dinv_ref, y_ref, r1_ref, cs_ref, s1_ref, cv_ref):
    i = pl.program_id(0)
    dinv = dinv_ref[...]
    r1 = r1_ref[0] + r1_ref[1]
    s1_ref[...] = dinv * (r1 + y_ref[...])
    csum = cs_ref[0] + cs_ref[1]
    cvec = dinv * (csum + dinv)
    row = lax.broadcasted_iota(jnp.int32, (rb, 128), 0)
    col = lax.broadcasted_iota(jnp.int32, (rb, 128), 1)
    gidx = i * blk + row * 128 + col
    cv_ref[...] = jnp.where(gidx < n, cvec, 0.0)

  return pl.pallas_call(
      body,
      grid=grid,
      in_specs=[
          pl.BlockSpec((rb, 128), lambda i: (i, 0)),
          pl.BlockSpec((rb, 128), lambda i: (i, 0)),
          pl.BlockSpec((2, rb, 128), lambda i: (0, i, 0)),
          pl.BlockSpec((2, rb, 128), lambda i: (0, i, 0)),
      ],
      out_specs=[
          pl.BlockSpec((rb, 128), lambda i: (i, 0)),
          pl.BlockSpec((rb, 128), lambda i: (i, 0)),
      ],
      out_shape=[
          jax.ShapeDtypeStruct((r, 128), jnp.float32),
          jax.ShapeDtypeStruct((r, 128), jnp.float32),
      ],
  )(dinv2, y2, r1p, csp)


def _tc_expand(s1n, dinvn, a1, c1, W2, npad):
  blk = 2048
  grid = (npad // blk,)

  def body(s1_ref, dinv_ref, a1_ref, c1_ref, w2_ref, *z_refs):
    h1 = jnp.maximum(s1_ref[...] * a1_ref[...] + c1_ref[...], 0.0)
    z = jnp.dot(h1, w2_ref[...], preferred_element_type=jnp.float32)
    zz = dinv_ref[...] * z
    for g in range(8):
      z_refs[g][...] = zz[:, 8 * g:8 * g + 8]

  return pl.pallas_call(
      body,
      grid=grid,
      in_specs=[
          pl.BlockSpec((blk, 1), lambda i: (i, 0)),
          pl.BlockSpec((blk, 1), lambda i: (i, 0)),
          pl.BlockSpec((1, 64), lambda i: (0, 0)),
          pl.BlockSpec((1, 64), lambda i: (0, 0)),
          pl.BlockSpec((64, 64), lambda i: (0, 0)),
      ],
      out_specs=[pl.BlockSpec((blk, 8), lambda i: (i, 0))] * 8,
      out_shape=[jax.ShapeDtypeStruct((npad, 8), jnp.float32)
                 for _ in range(8)],
  )(s1n, dinvn, a1, c1, W2)


def _tc_final(r2g, zzg, dinvn, cvn, gs2, cb2, W3, a3, cc3,
              Wl1, bl1, Wl2, bl2, npad):
  blk = 2048
  grid = (npad // blk,)
  last = npad // blk - 1

  def body(*refs):
    r_refs = refs[:8]
    z_refs = refs[8:16]
    (dinv_ref, cv_ref, gs2_ref, cb2_ref, w3_ref, a3_ref, cc3_ref,
     wl1_ref, bl1_ref, wl2_ref, bl2_ref, out_ref, acc_ref) = refs[16:]
    i = pl.program_id(0)
    rr = jnp.concatenate([r[...] for r in r_refs], axis=1)
    zz = jnp.concatenate([z[...] for z in z_refs], axis=1)
    out2 = dinv_ref[...] * (rr + zz)
    h2 = jnp.maximum(out2 * gs2_ref[...] + cb2_ref[...], 0.0)
    pv = lax.dot_general(cv_ref[...], h2, (((0,), (0,)), ((), ())),
                         preferred_element_type=jnp.float32)

    @pl.when(i == 0)
    def _():
      acc_ref[...] = jnp.zeros_like(acc_ref)

    acc_ref[...] += pv

    @pl.when(i == last)
    def _():
      v = acc_ref[...]
      m = jnp.dot(v, w3_ref[...], preferred_element_type=jnp.float32)
      m = m * a3_ref[...] + cc3_ref[...]
      h = jnp.maximum(
          jnp.dot(m, wl1_ref[...], preferred_element_type=jnp.float32)
          + bl1_ref[...], 0.0)
      logits = jnp.dot(h, wl2_ref[...],
                       preferred_element_type=jnp.float32) + bl2_ref[...]
      mx = jnp.max(logits, axis=1, keepdims=True)
      ex = jnp.exp(logits - mx)
      out_ref[...] = ex / jnp.sum(ex, axis=1, keepdims=True)

  return pl.pallas_call(
      body,
      grid=grid,
      in_specs=[pl.BlockSpec((blk, 8), lambda i: (i, 0))] * 16
      + [
          pl.BlockSpec((blk, 1), lambda i: (i, 0)),
          pl.BlockSpec((blk, 1), lambda i: (i, 0)),
          pl.BlockSpec((1, 64), lambda i: (0, 0)),
          pl.BlockSpec((1, 64), lambda i: (0, 0)),
          pl.BlockSpec((64, 64), lambda i: (0, 0)),
          pl.BlockSpec((1, 64), lambda i: (0, 0)),
          pl.BlockSpec((1, 64), lambda i: (0, 0)),
          pl.BlockSpec((64, 32), lambda i: (0, 0)),
          pl.BlockSpec((1, 32), lambda i: (0, 0)),
          pl.BlockSpec((32, 3), lambda i: (0, 0)),
          pl.BlockSpec((1, 3), lambda i: (0, 0)),
      ],
      out_specs=pl.BlockSpec((1, 3), lambda i: (0, 0)),
      out_shape=jax.ShapeDtypeStruct((1, 3), jnp.float32),
      scratch_shapes=[pltpu.VMEM((1, 64), jnp.float32)],
  )(*r2g, *zzg, dinvn, cvn, gs2, cb2, W3, a3, cc3, Wl1, bl1, Wl2, bl2)


def kernel(x, edge_index, W1, b1, g1, be1, W2, b2, g2, be2, W3, b3, g3, be3,
           Wl1, bl1, Wl2, bl2):
  n = x.shape[0]
  e = edge_index.shape[1]
  npad, rows = _pad_sizes(n, e)
  epad = rows * 128

  xp = jnp.zeros((npad,), jnp.float32).at[:n].set(x[:, 0])
  pad_idx = (n + (jnp.arange(epad - e, dtype=jnp.int32) % 256))
  src = jnp.concatenate([edge_index[0].astype(jnp.int32), pad_idx])
  dst = jnp.concatenate([edge_index[1].astype(jnp.int32), pad_idx])
  src2d = src.reshape(rows, 128)
  dst2d = dst.reshape(rows, 128)
  xp2 = xp.reshape(npad // 128, 128)

  gs1 = g1 / jnp.sqrt(1.0 + _EPS)
  gs2 = g2 / jnp.sqrt(1.0 + _EPS)
  gs3 = g3 / jnp.sqrt(1.0 + _EPS)
  a1 = (W1[0] * gs1).reshape(1, 64)
  c1 = (b1 * gs1 + be1).reshape(1, 64)
  cb2 = (b2 * gs2 + be2).reshape(1, 64)
  a3 = (gs3 / np.float32(n)).reshape(1, 64)
  cc3 = (b3 * gs3 + be3).reshape(1, 64)

  degp = _sc_degree(dst2d, npad, rows)
  degp = degp.reshape(_NC, npad // 128, 128)
  dinv2, y2 = _tc_prep(degp, xp2, npad)

  r1p, csp = _sc_scalar_pass(src2d, dst2d, y2.reshape(npad),
                             dinv2.reshape(npad), npad, rows)
  r1p = r1p.reshape(_NC, npad // 128, 128)
  csp = csp.reshape(_NC, npad // 128, 128)
  s1_2d, cv2 = _tc_s1(dinv2, y2, r1p, csp, n, npad)
  dinvn = dinv2.reshape(npad, 1)
  zzg = _tc_expand(s1_2d.reshape(npad, 1), dinvn, a1, c1, W2, npad)

  zeros8 = jnp.zeros((128, 8), jnp.float32)
  r2g = _sc_wide_pass(src2d, dst2d, zzg, zeros8, npad, rows)
  out = _tc_final(r2g, zzg, dinvn, cv2.reshape(npad, 1),
                  gs2.reshape(1, 64), cb2, W3, a3, cc3, Wl1,
                  bl1.reshape(1, 32), Wl2.reshape(32, 3),
                  bl2.reshape(1, 3), npad)
  return out


# speedup vs baseline: 40.1200x; 1.5033x over previous
import functools

import jax
import jax.numpy as jnp
import numpy as np
from jax import lax
from jax.experimental import pallas as pl
from jax.experimental.pallas import tpu as pltpu
from jax.experimental.pallas import tpu_sc as plsc

_EPS = 1e-5

_NC = 2
_NS = 16
_NW = _NC * _NS
_CH = 16


def _pad_sizes(n, e):
  npad = ((n + 256 + 2047) // 2048) * 2048
  rows = -(-e // 128)
  rows = -(-rows // 512) * 512
  return npad, rows


def _row_block(r):
  return 56 if r % 56 == 0 else 16


def _mesh():
  return plsc.VectorSubcoreMesh(core_axis_name="c", subcore_axis_name="s",
                                num_cores=_NC, num_subcores=_NS)


def _sc_degree(dst2d, npad, rows):
  sl = npad // _NS
  rpw = rows // _NW
  chunks = rpw // _CH

  @functools.partial(
      pl.kernel,
      out_type=jax.ShapeDtypeStruct((_NW * sl,), jnp.float32),
      mesh=_mesh(),
      scratch_types=[
          pltpu.VMEM((_CH, 128), jnp.int32),
          pltpu.VMEM((128,), jnp.float32),
          pltpu.VMEM((sl,), jnp.float32),
          pltpu.VMEM_SHARED((npad,), jnp.float32),
          pltpu.SemaphoreType.DMA,
      ],
  )
  def k(dst_hbm, out_hbm, idx_v, ones_v, slice_v, acc_sp, sem):
    c = lax.axis_index("c")
    s = lax.axis_index("s")
    w = c * _NS + s

    @pl.loop(0, 8)
    def _(i):
      ones_v[pl.ds(i * 16, 16)] = jnp.ones((16,), jnp.float32)

    @pl.loop(0, sl // 16)
    def _(i):
      slice_v[pl.ds(i * 16, 16)] = jnp.zeros((16,), jnp.float32)

    pltpu.sync_copy(slice_v, acc_sp.at[pl.ds(s * sl, sl)])
    plsc.subcore_barrier()

    @pl.loop(0, chunks)
    def _(t):
      base = w * rpw + t * _CH
      pltpu.sync_copy(dst_hbm.at[pl.ds(base, _CH)], idx_v)
      for j in range(_CH):
        pltpu.async_copy(ones_v, acc_sp.at[idx_v.at[j]], sem, add=True)
      pltpu.make_async_copy(dst_hbm.at[pl.ds(0, _CH)], idx_v, sem).wait()

    plsc.subcore_barrier()
    pltpu.sync_copy(acc_sp.at[pl.ds(s * sl, sl)], slice_v)
    pltpu.sync_copy(slice_v, out_hbm.at[pl.ds(w * sl, sl)])

  return k(dst2d)


def _sc_scalar_pass(src2d, dst2d, y_hbm, dinv_hbm, npad, rows):
  sl = npad // _NS
  rpw = rows // _NW
  chunks = rpw // _CH

  @functools.partial(
      pl.kernel,
      out_type=[
          jax.ShapeDtypeStruct((_NW * sl,), jnp.float32),
          jax.ShapeDtypeStruct((_NW * sl,), jnp.float32),
      ],
      mesh=_mesh(),
      scratch_types=[
          pltpu.VMEM((_CH, 128), jnp.int32),
          pltpu.VMEM((_CH, 128), jnp.int32),
          pltpu.VMEM((_CH, 128), jnp.float32),
          pltpu.VMEM((_CH, 128), jnp.float32),
          pltpu.VMEM((sl,), jnp.float32),
          pltpu.VMEM_SHARED((npad,), jnp.float32),
          pltpu.VMEM_SHARED((npad,), jnp.float32),
          pltpu.VMEM_SHARED((npad,), jnp.float32),
          pltpu.VMEM_SHARED((npad,), jnp.float32),
          pltpu.SemaphoreType.DMA,
          pltpu.SemaphoreType.DMA,
      ],
  )
  def k(src_hbm, dst_hbm, y_in, dinv_in, r1_out, cs_out,
        idx_s, idx_d, yv, dv, slice_v, y_sp, dinv_sp, r1_sp, cs_sp,
        gsem, ssem):
    c = lax.axis_index("c")
    s = lax.axis_index("s")
    w = c * _NS + s

    pltpu.sync_copy(y_in.at[pl.ds(s * sl, sl)], slice_v)
    pltpu.sync_copy(slice_v, y_sp.at[pl.ds(s * sl, sl)])
    pltpu.sync_copy(dinv_in.at[pl.ds(s * sl, sl)], slice_v)
    pltpu.sync_copy(slice_v, dinv_sp.at[pl.ds(s * sl, sl)])

    @pl.loop(0, sl // 16)
    def _(i):
      slice_v[pl.ds(i * 16, 16)] = jnp.zeros((16,), jnp.float32)

    pltpu.sync_copy(slice_v, r1_sp.at[pl.ds(s * sl, sl)])
    pltpu.sync_copy(slice_v, cs_sp.at[pl.ds(s * sl, sl)])
    plsc.subcore_barrier()

    @pl.loop(0, chunks)
    def _(t):
      base = w * rpw + t * _CH
      pltpu.sync_copy(src_hbm.at[pl.ds(base, _CH)], idx_s)
      pltpu.sync_copy(dst_hbm.at[pl.ds(base, _CH)], idx_d)
      for j in range(_CH):
        pltpu.async_copy(y_sp.at[idx_s.at[j]], yv.at[j], gsem)
        pltpu.async_copy(dinv_sp.at[idx_d.at[j]], dv.at[j], gsem)
      pltpu.make_async_copy(src_hbm.at[pl.ds(0, _CH)], yv, gsem).wait()
      pltpu.make_async_copy(src_hbm.at[pl.ds(0, _CH)], dv, gsem).wait()
      for j in range(_CH):
        pltpu.async_copy(yv.at[j], r1_sp.at[idx_d.at[j]], ssem, add=True)
        pltpu.async_copy(dv.at[j], cs_sp.at[idx_s.at[j]], ssem, add=True)
      pltpu.make_async_copy(src_hbm.at[pl.ds(0, _CH)], yv, ssem).wait()
      pltpu.make_async_copy(src_hbm.at[pl.ds(0, _CH)], dv, ssem).wait()

    plsc.subcore_barrier()
    pltpu.sync_copy(r1_sp.at[pl.ds(s * sl, sl)], slice_v)
    pltpu.sync_copy(slice_v, r1_out.at[pl.ds(w * sl, sl)])
    pltpu.sync_copy(cs_sp.at[pl.ds(s * sl, sl)], slice_v)
    pltpu.sync_copy(slice_v, cs_out.at[pl.ds(w * sl, sl)])

  return k(src2d, dst2d, y_hbm, dinv_hbm)


def _sc_wide_pass(il2, zz, zeros16, npad, rows):
  sl = npad // _NS
  rpw = rows // _NS
  nsc = rpw // 16
  assert nsc % 2 == 0

  @functools.partial(
      pl.kernel,
      out_type=[jax.ShapeDtypeStruct((npad, 16), jnp.float32)
                for _ in range(4)],
      mesh=_mesh(),
      compiler_params=pltpu.CompilerParams(use_tc_tiling_on_sc=False),
      scratch_types=[
          pltpu.VMEM((16, 2, 128), jnp.int32),
          pltpu.VMEM((16, 2, 128), jnp.int32),
          pltpu.VMEM((512, 16), jnp.float32),
          pltpu.VMEM((512, 16), jnp.float32),
          pltpu.VMEM_SHARED((npad, 16), jnp.float32),
          pltpu.SemaphoreType.DMA,
          pltpu.SemaphoreType.DMA,
          pltpu.SemaphoreType.DMA,
          pltpu.SemaphoreType.DMA,
      ],
  )
  def k(il_hbm, zeros_hbm, zz0, zz1, zz2, zz3, o0, o1, o2, o3,
        il0, il1, rv0, rv1, acc_sp, gsem0, gsem1, ssem0, ssem1):
    zzs = [zz0, zz1, zz2, zz3]
    outs = [o0, o1, o2, o3]
    il = [il0, il1]
    rv = [rv0, rv1]
    gsem = [gsem0, gsem1]
    ssem = [ssem0, ssem1]
    c = lax.axis_index("c")
    s = lax.axis_index("s")

    for g in range(4):
      @pl.when(c == g // 2)
      def _():
        zz_g = zzs[g]
        out_g = outs[g]

        def stage(sc, p):
          pltpu.sync_copy(il_hbm.at[pl.ds(s * rpw + sc * 16, 16)], il[p])

        def fire_gathers(p, sub, b):
          for kk in range(4):
            pltpu.async_copy(zz_g.at[il[p].at[4 * sub + kk, 0]],
                             rv[b].at[pl.ds(kk * 128, 128)], gsem[b])

        def fire_scatters(p, sub, b):
          for kk in range(4):
            pltpu.async_copy(rv[b].at[pl.ds(kk * 128, 128)],
                             acc_sp.at[il[p].at[4 * sub + kk, 1]], ssem[b],
                             add=True)

        def drain(sem, b):
          pltpu.make_async_copy(zz_g.at[pl.ds(0, 512)], rv[b], sem).wait()

        pltpu.sync_copy(zeros_hbm, rv0.at[pl.ds(0, 128)])

        @pl.loop(0, sl // 128)
        def _(i):
          pltpu.sync_copy(rv0.at[pl.ds(0, 128)],
                          acc_sp.at[pl.ds(s * sl + i * 128, 128)])

        plsc.subcore_barrier()

        pltpu.async_copy(zz_g.at[pl.ds(0, 512)], rv0, ssem0)
        pltpu.async_copy(zz_g.at[pl.ds(0, 512)], rv1, ssem1)
        stage(0, 0)
        drain(ssem[0], 0)
        fire_gathers(0, 0, 0)

        @pl.loop(0, nsc, step=2)
        def _(t):
          for p in range(2):
            sc = t + p

            for sub in range(4):
              b = sub % 2
              nb = 1 - b
              drain(ssem[nb], nb)
              if sub == 0:
                @pl.when(sc + 1 < nsc)
                def _():
                  stage(sc + 1, 1 - p)
              if sub < 3:
                fire_gathers(p, sub + 1, nb)
              else:
                @pl.when(sc + 1 < nsc)
                def _():
                  fire_gathers(1 - p, 0, nb)
              drain(gsem[b], b)
              fire_scatters(p, sub, b)

        drain(ssem[1], 1)
        plsc.subcore_barrier()

        @pl.loop(0, sl // 128)
        def _(i):
          pltpu.sync_copy(acc_sp.at[pl.ds(s * sl + i * 128, 128)],
                          rv0.at[pl.ds(0, 128)])
          pltpu.sync_copy(rv0.at[pl.ds(0, 128)],
                          out_g.at[pl.ds(s * sl + i * 128, 128)])

  return k(il2, zeros16, *zz)


def _tc_prep(degp, xp2, npad):
  r = npad // 128
  rb = _row_block(r)
  grid = (r // rb,)

  def body(degp_ref, x_ref, dinv_ref, y_ref):
    deg = degp_ref[0] + degp_ref[1] + 1.0
    dinv = lax.rsqrt(deg)
    dinv_ref[...] = dinv
    y_ref[...] = dinv * x_ref[...]

  return pl.pallas_call(
      body,
      grid=grid,
      in_specs=[
          pl.BlockSpec((2, rb, 128), lambda i: (0, i, 0)),
          pl.BlockSpec((rb, 128), lambda i: (i, 0)),
      ],
      out_specs=[
          pl.BlockSpec((rb, 128), lambda i: (i, 0)),
          pl.BlockSpec((rb, 128), lambda i: (i, 0)),
      ],
      out_shape=[
          jax.ShapeDtypeStruct((r, 128), jnp.float32),
          jax.ShapeDtypeStruct((r, 128), jnp.float32),
      ],
  )(degp, xp2)


def _tc_s1(dinv2, y2, r1p, csp, n, npad):
  r = npad // 128
  rb = _row_block(r)
  blk = rb * 128
  grid = (r // rb,)

  def body(dinv_ref, y_ref, r1_ref, cs_ref, s1_ref, cv_ref):
    i = pl.program_id(0)
    dinv = dinv_ref[...]
    r1 = r1_ref[0] + r1_ref[1]
    s1_ref[...] = dinv * (r1 + y_ref[...])
    csum = cs_ref[0] + cs_ref[1]
    cvec = dinv * (csum + dinv)
    row = lax.broadcasted_iota(jnp.int32, (rb, 128), 0)
    col = lax.broadcasted_iota(jnp.int32, (rb, 128), 1)
    gidx = i * blk + row * 128 + col
    cv_ref[...] = jnp.where(gidx < n, cvec, 0.0)

  return pl.pallas_call(
      body,
      grid=grid,
      in_specs=[
          pl.BlockSpec((rb, 128), lambda i: (i, 0)),
          pl.BlockSpec((rb, 128), lambda i: (i, 0)),
          pl.BlockSpec((2, rb, 128), lambda i: (0, i, 0)),
          pl.BlockSpec((2, rb, 128), lambda i: (0, i, 0)),
      ],
      out_specs=[
          pl.BlockSpec((rb, 128), lambda i: (i, 0)),
          pl.BlockSpec((rb, 128), lambda i: (i, 0)),
      ],
      out_shape=[
          jax.ShapeDtypeStruct((r, 128), jnp.float32),
          jax.ShapeDtypeStruct((r, 128), jnp.float32),
      ],
  )(dinv2, y2, r1p, csp)


def _tc_expand(s1n, dinvn, a1, c1, W2, npad):
  blk = 2048
  grid = (npad // blk,)

  def body(s1_ref, dinv_ref, a1_ref, c1_ref, w2_ref,
           z0_ref, z1_ref, z2_ref, z3_ref, dg_ref):
    z_refs = [z0_ref, z1_ref, z2_ref, z3_ref]
    dinv = dinv_ref[...]
    h1 = jnp.maximum(s1_ref[...] * a1_ref[...] + c1_ref[...], 0.0)
    z = jnp.dot(h1, w2_ref[...], preferred_element_type=jnp.float32)
    zz = dinv * z
    for g in range(4):
      z_refs[g][...] = zz[:, 16 * g:16 * g + 16]
    dg_ref[...] = jnp.broadcast_to(dinv, (blk, 16))

  return pl.pallas_call(
      body,
      grid=grid,
      in_specs=[
          pl.BlockSpec((blk, 1), lambda i: (i, 0)),
          pl.BlockSpec((blk, 1), lambda i: (i, 0)),
          pl.BlockSpec((1, 64), lambda i: (0, 0)),
          pl.BlockSpec((1, 64), lambda i: (0, 0)),
          pl.BlockSpec((64, 64), lambda i: (0, 0)),
      ],
      out_specs=[pl.BlockSpec((blk, 16), lambda i: (i, 0))] * 5,
      out_shape=[jax.ShapeDtypeStruct((npad, 16), jnp.float32)
                 for _ in range(5)],
  )(s1n, dinvn, a1, c1, W2)


def _tc_final(r2g, zzg, dinvg, cvn, gs2, cb2, W3, a3, cc3,
              Wl1, bl1, Wl2, bl2, npad):
  blk = 2048
  grid = (npad // blk,)
  last = npad // blk - 1

  def body(*refs):
    r_refs = refs[:4]
    z_refs = refs[4:8]
    (dg_ref, cv_ref, gs2_ref, cb2_ref, w3_ref, a3_ref, cc3_ref,
     wl1_ref, bl1_ref, wl2_ref, bl2_ref, out_ref, acc_ref) = refs[8:]
    i = pl.program_id(0)

    @pl.when(i == 0)
    def _():
      acc_ref[...] = jnp.zeros_like(acc_ref)

    dg = dg_ref[...]
    cv = cv_ref[...]
    for g in range(4):
      out2 = dg * (r_refs[g][...] + z_refs[g][...])
      h2 = jnp.maximum(out2 * gs2_ref[0:1, 16 * g:16 * g + 16]
                       + cb2_ref[0:1, 16 * g:16 * g + 16], 0.0)
      pv = lax.dot_general(cv, h2, (((0,), (0,)), ((), ())),
                           preferred_element_type=jnp.float32)
      acc_ref[0:1, 16 * g:16 * g + 16] += pv

    @pl.when(i == last)
    def _():
      v = acc_ref[...]
      m = jnp.dot(v, w3_ref[...], preferred_element_type=jnp.float32)
      m = m * a3_ref[...] + cc3_ref[...]
      h = jnp.maximum(
          jnp.dot(m, wl1_ref[...], preferred_element_type=jnp.float32)
          + bl1_ref[...], 0.0)
      logits = jnp.dot(h, wl2_ref[...],
                       preferred_element_type=jnp.float32) + bl2_ref[...]
      mx = jnp.max(logits, axis=1, keepdims=True)
      ex = jnp.exp(logits - mx)
      out_ref[...] = ex / jnp.sum(ex, axis=1, keepdims=True)

  return pl.pallas_call(
      body,
      grid=grid,
      in_specs=[pl.BlockSpec((blk, 16), lambda i: (i, 0))] * 9
      + [
          pl.BlockSpec((blk, 1), lambda i: (i, 0)),
          pl.BlockSpec((1, 64), lambda i: (0, 0)),
          pl.BlockSpec((1, 64), lambda i: (0, 0)),
          pl.BlockSpec((64, 64), lambda i: (0, 0)),
          pl.BlockSpec((1, 64), lambda i: (0, 0)),
          pl.BlockSpec((1, 64), lambda i: (0, 0)),
          pl.BlockSpec((64, 32), lambda i: (0, 0)),
          pl.BlockSpec((1, 32), lambda i: (0, 0)),
          pl.BlockSpec((32, 3), lambda i: (0, 0)),
          pl.BlockSpec((1, 3), lambda i: (0, 0)),
      ],
      out_specs=pl.BlockSpec((1, 3), lambda i: (0, 0)),
      out_shape=jax.ShapeDtypeStruct((1, 3), jnp.float32),
      scratch_shapes=[pltpu.VMEM((1, 64), jnp.float32)],
  )(*r2g, *zzg, dinvg, cvn, gs2, cb2, W3, a3, cc3, Wl1, bl1, Wl2, bl2)


def kernel(x, edge_index, W1, b1, g1, be1, W2, b2, g2, be2, W3, b3, g3, be3,
           Wl1, bl1, Wl2, bl2):
  n = x.shape[0]
  e = edge_index.shape[1]
  npad, rows = _pad_sizes(n, e)
  epad = rows * 128

  xp = jnp.zeros((npad,), jnp.float32).at[:n].set(x[:, 0])
  pad_idx = (n + (jnp.arange(epad - e, dtype=jnp.int32) % 256))
  src = jnp.concatenate([edge_index[0].astype(jnp.int32), pad_idx])
  dst = jnp.concatenate([edge_index[1].astype(jnp.int32), pad_idx])
  src2d = src.reshape(rows, 128)
  dst2d = dst.reshape(rows, 128)
  xp2 = xp.reshape(npad // 128, 128)

  gs1 = g1 / jnp.sqrt(1.0 + _EPS)
  gs2 = g2 / jnp.sqrt(1.0 + _EPS)
  gs3 = g3 / jnp.sqrt(1.0 + _EPS)
  a1 = (W1[0] * gs1).reshape(1, 64)
  c1 = (b1 * gs1 + be1).reshape(1, 64)
  cb2 = (b2 * gs2 + be2).reshape(1, 64)
  a3 = (gs3 / np.float32(n)).reshape(1, 64)
  cc3 = (b3 * gs3 + be3).reshape(1, 64)

  degp = _sc_degree(dst2d, npad, rows)
  degp = degp.reshape(_NC, npad // 128, 128)
  dinv2, y2 = _tc_prep(degp, xp2, npad)

  r1p, csp = _sc_scalar_pass(src2d, dst2d, y2.reshape(npad),
                             dinv2.reshape(npad), npad, rows)
  r1p = r1p.reshape(_NC, npad // 128, 128)
  csp = csp.reshape(_NC, npad // 128, 128)
  s1_2d, cv2 = _tc_s1(dinv2, y2, r1p, csp, n, npad)
  dinvn = dinv2.reshape(npad, 1)
  exp_out = _tc_expand(s1_2d.reshape(npad, 1), dinvn, a1, c1, W2, npad)
  zzg, dinvg = exp_out[:4], exp_out[4]

  il2 = jnp.stack([src2d, dst2d], axis=1)
  zeros16 = jnp.zeros((128, 16), jnp.float32)
  r2g = _sc_wide_pass(il2, zzg, zeros16, npad, rows)
  out = _tc_final(r2g, zzg, dinvg, cv2.reshape(npad, 1),
                  gs2.reshape(1, 64), cb2, W3, a3, cc3, Wl1,
                  bl1.reshape(1, 32), Wl2.reshape(32, 3),
                  bl2.reshape(1, 3), npad)
  return out


# speedup vs baseline: 43.3696x; 1.0810x over previous
import functools

import jax
import jax.numpy as jnp
import numpy as np
from jax import lax
from jax.experimental import pallas as pl
from jax.experimental.pallas import tpu as pltpu
from jax.experimental.pallas import tpu_sc as plsc

_EPS = 1e-5

_NC = 2
_NS = 16
_NW = _NC * _NS
_CH = 16


def _pad_sizes(n, e):
  npad = ((n + 256 + 2047) // 2048) * 2048
  rows = -(-e // 128)
  rows = -(-rows // 512) * 512
  return npad, rows


def _row_block(r):
  return 56 if r % 56 == 0 else 16


def _mesh():
  return plsc.VectorSubcoreMesh(core_axis_name="c", subcore_axis_name="s",
                                num_cores=_NC, num_subcores=_NS)


def _sc_degree(dst2d, npad, rows):
  sl = npad // _NS
  rpw = rows // _NW
  chunks = rpw // _CH

  @functools.partial(
      pl.kernel,
      out_type=jax.ShapeDtypeStruct((_NW * sl,), jnp.float32),
      mesh=_mesh(),
      scratch_types=[
          pltpu.VMEM((_CH, 128), jnp.int32),
          pltpu.VMEM((128,), jnp.float32),
          pltpu.VMEM((sl,), jnp.float32),
          pltpu.VMEM_SHARED((npad,), jnp.float32),
          pltpu.SemaphoreType.DMA,
      ],
  )
  def k(dst_hbm, out_hbm, idx_v, ones_v, slice_v, acc_sp, sem):
    c = lax.axis_index("c")
    s = lax.axis_index("s")
    w = c * _NS + s

    @pl.loop(0, 8)
    def _(i):
      ones_v[pl.ds(i * 16, 16)] = jnp.ones((16,), jnp.float32)

    @pl.loop(0, sl // 16)
    def _(i):
      slice_v[pl.ds(i * 16, 16)] = jnp.zeros((16,), jnp.float32)

    pltpu.sync_copy(slice_v, acc_sp.at[pl.ds(s * sl, sl)])
    plsc.subcore_barrier()

    @pl.loop(0, chunks)
    def _(t):
      base = w * rpw + t * _CH
      pltpu.sync_copy(dst_hbm.at[pl.ds(base, _CH)], idx_v)
      for j in range(_CH):
        pltpu.async_copy(ones_v, acc_sp.at[idx_v.at[j]], sem, add=True)
      pltpu.make_async_copy(dst_hbm.at[pl.ds(0, _CH)], idx_v, sem).wait()

    plsc.subcore_barrier()
    pltpu.sync_copy(acc_sp.at[pl.ds(s * sl, sl)], slice_v)
    pltpu.sync_copy(slice_v, out_hbm.at[pl.ds(w * sl, sl)])

  return k(dst2d)


def _sc_scalar_pass(src2d, dst2d, y_hbm, dinv_hbm, npad, rows):
  sl = npad // _NS
  rpw = rows // _NW
  chunks = rpw // _CH

  @functools.partial(
      pl.kernel,
      out_type=[
          jax.ShapeDtypeStruct((_NW * sl,), jnp.float32),
          jax.ShapeDtypeStruct((_NW * sl,), jnp.float32),
      ],
      mesh=_mesh(),
      scratch_types=[
          pltpu.VMEM((_CH, 128), jnp.int32),
          pltpu.VMEM((_CH, 128), jnp.int32),
          pltpu.VMEM((_CH, 128), jnp.float32),
          pltpu.VMEM((_CH, 128), jnp.float32),
          pltpu.VMEM((sl,), jnp.float32),
          pltpu.VMEM_SHARED((npad,), jnp.float32),
          pltpu.VMEM_SHARED((npad,), jnp.float32),
          pltpu.VMEM_SHARED((npad,), jnp.float32),
          pltpu.VMEM_SHARED((npad,), jnp.float32),
          pltpu.SemaphoreType.DMA,
          pltpu.SemaphoreType.DMA,
      ],
  )
  def k(src_hbm, dst_hbm, y_in, dinv_in, r1_out, cs_out,
        idx_s, idx_d, yv, dv, slice_v, y_sp, dinv_sp, r1_sp, cs_sp,
        gsem, ssem):
    c = lax.axis_index("c")
    s = lax.axis_index("s")
    w = c * _NS + s

    pltpu.sync_copy(y_in.at[pl.ds(s * sl, sl)], slice_v)
    pltpu.sync_copy(slice_v, y_sp.at[pl.ds(s * sl, sl)])
    pltpu.sync_copy(dinv_in.at[pl.ds(s * sl, sl)], slice_v)
    pltpu.sync_copy(slice_v, dinv_sp.at[pl.ds(s * sl, sl)])

    @pl.loop(0, sl // 16)
    def _(i):
      slice_v[pl.ds(i * 16, 16)] = jnp.zeros((16,), jnp.float32)

    pltpu.sync_copy(slice_v, r1_sp.at[pl.ds(s * sl, sl)])
    pltpu.sync_copy(slice_v, cs_sp.at[pl.ds(s * sl, sl)])
    plsc.subcore_barrier()

    @pl.loop(0, chunks)
    def _(t):
      base = w * rpw + t * _CH
      pltpu.sync_copy(src_hbm.at[pl.ds(base, _CH)], idx_s)
      pltpu.sync_copy(dst_hbm.at[pl.ds(base, _CH)], idx_d)
      for j in range(_CH):
        pltpu.async_copy(y_sp.at[idx_s.at[j]], yv.at[j], gsem)
        pltpu.async_copy(dinv_sp.at[idx_d.at[j]], dv.at[j], gsem)
      pltpu.make_async_copy(src_hbm.at[pl.ds(0, _CH)], yv, gsem).wait()
      pltpu.make_async_copy(src_hbm.at[pl.ds(0, _CH)], dv, gsem).wait()
      for j in range(_CH):
        pltpu.async_copy(yv.at[j], r1_sp.at[idx_d.at[j]], ssem, add=True)
        pltpu.async_copy(dv.at[j], cs_sp.at[idx_s.at[j]], ssem, add=True)
      pltpu.make_async_copy(src_hbm.at[pl.ds(0, _CH)], yv, ssem).wait()
      pltpu.make_async_copy(src_hbm.at[pl.ds(0, _CH)], dv, ssem).wait()

    plsc.subcore_barrier()
    pltpu.sync_copy(r1_sp.at[pl.ds(s * sl, sl)], slice_v)
    pltpu.sync_copy(slice_v, r1_out.at[pl.ds(w * sl, sl)])
    pltpu.sync_copy(cs_sp.at[pl.ds(s * sl, sl)], slice_v)
    pltpu.sync_copy(slice_v, cs_out.at[pl.ds(w * sl, sl)])

  return k(src2d, dst2d, y_hbm, dinv_hbm)


def _sc_wide_pass(il2, zz, zeros16, npad, rows):
  sl = npad // _NS
  rpw = rows // _NS
  nsc = rpw // 16
  assert nsc % 2 == 0

  @functools.partial(
      pl.kernel,
      out_type=[jax.ShapeDtypeStruct((npad, 16), jnp.float32)
                for _ in range(2)],
      mesh=_mesh(),
      compiler_params=pltpu.CompilerParams(use_tc_tiling_on_sc=False),
      scratch_types=[
          pltpu.VMEM((16, 2, 128), jnp.int32),
          pltpu.VMEM((16, 2, 128), jnp.int32),
          pltpu.VMEM((512, 16), jnp.float32),
          pltpu.VMEM((512, 16), jnp.float32),
          pltpu.VMEM_SHARED((npad, 16), jnp.float32),
          pltpu.SemaphoreType.DMA,
          pltpu.SemaphoreType.DMA,
          pltpu.SemaphoreType.DMA,
          pltpu.SemaphoreType.DMA,
      ],
  )
  def k(il_hbm, zeros_hbm, zz0, zz1, o0, o1,
        il0, il1, rv0, rv1, acc_sp, gsem0, gsem1, ssem0, ssem1):
    zzs = [zz0, zz1]
    outs = [o0, o1]
    il = [il0, il1]
    rv = [rv0, rv1]
    gsem = [gsem0, gsem1]
    ssem = [ssem0, ssem1]
    c = lax.axis_index("c")
    s = lax.axis_index("s")

    for g in range(2):
      @pl.when(c == g)
      def _():
        zz_g = zzs[g]
        out_g = outs[g]

        def stage(sc, p):
          pltpu.sync_copy(il_hbm.at[pl.ds(s * rpw + sc * 16, 16)], il[p])

        def fire_gathers(p, sub, b):
          for kk in range(4):
            pltpu.async_copy(zz_g.at[il[p].at[4 * sub + kk, 0]],
                             rv[b].at[pl.ds(kk * 128, 128)], gsem[b])

        def fire_scatters(p, sub, b):
          for kk in range(4):
            pltpu.async_copy(rv[b].at[pl.ds(kk * 128, 128)],
                             acc_sp.at[il[p].at[4 * sub + kk, 1]], ssem[b],
                             add=True)

        def drain(sem, b):
          pltpu.make_async_copy(zz_g.at[pl.ds(0, 512)], rv[b], sem).wait()

        pltpu.sync_copy(zeros_hbm, rv0.at[pl.ds(0, 128)])

        @pl.loop(0, sl // 128)
        def _(i):
          pltpu.sync_copy(rv0.at[pl.ds(0, 128)],
                          acc_sp.at[pl.ds(s * sl + i * 128, 128)])

        plsc.subcore_barrier()

        pltpu.async_copy(zz_g.at[pl.ds(0, 512)], rv0, ssem0)
        pltpu.async_copy(zz_g.at[pl.ds(0, 512)], rv1, ssem1)
        stage(0, 0)
        drain(ssem[0], 0)
        fire_gathers(0, 0, 0)

        @pl.loop(0, nsc, step=2)
        def _(t):
          for p in range(2):
            sc = t + p

            for sub in range(4):
              b = sub % 2
              nb = 1 - b
              drain(ssem[nb], nb)
              if sub == 0:
                @pl.when(sc + 1 < nsc)
                def _():
                  stage(sc + 1, 1 - p)
              if sub < 3:
                fire_gathers(p, sub + 1, nb)
              else:
                @pl.when(sc + 1 < nsc)
                def _():
                  fire_gathers(1 - p, 0, nb)
              drain(gsem[b], b)
              fire_scatters(p, sub, b)

        drain(ssem[1], 1)
        plsc.subcore_barrier()

        @pl.loop(0, sl // 128)
        def _(i):
          pltpu.sync_copy(acc_sp.at[pl.ds(s * sl + i * 128, 128)],
                          rv0.at[pl.ds(0, 128)])
          pltpu.sync_copy(rv0.at[pl.ds(0, 128)],
                          out_g.at[pl.ds(s * sl + i * 128, 128)])

  return k(il2, zeros16, zz[0], zz[1])


def _tc_prep(degp, xp2, npad):
  r = npad // 128
  rb = _row_block(r)
  grid = (r // rb,)

  def body(degp_ref, x_ref, dinv_ref, y_ref):
    deg = degp_ref[0] + degp_ref[1] + 1.0
    dinv = lax.rsqrt(deg)
    dinv_ref[...] = dinv
    y_ref[...] = dinv * x_ref[...]

  return pl.pallas_call(
      body,
      grid=grid,
      in_specs=[
          pl.BlockSpec((2, rb, 128), lambda i: (0, i, 0)),
          pl.BlockSpec((rb, 128), lambda i: (i, 0)),
      ],
      out_specs=[
          pl.BlockSpec((rb, 128), lambda i: (i, 0)),
          pl.BlockSpec((rb, 128), lambda i: (i, 0)),
      ],
      out_shape=[
          jax.ShapeDtypeStruct((r, 128), jnp.float32),
          jax.ShapeDtypeStruct((r, 128), jnp.float32),
      ],
  )(degp, xp2)


def _tc_s1(dinv2, y2, r1p, csp, n, npad):
  r = npad // 128
  rb = _row_block(r)
  blk = rb * 128
  grid = (r // rb,)

  def body(dinv_ref, y_ref, r1_ref, cs_ref, s1_ref, cv_ref):
    i = pl.program_id(0)
    dinv = dinv_ref[...]
    r1 = r1_ref[0] + r1_ref[1]
    s1_ref[...] = dinv * (r1 + y_ref[...])
    csum = cs_ref[0] + cs_ref[1]
    cvec = dinv * (csum + dinv)
    row = lax.broadcasted_iota(jnp.int32, (rb, 128), 0)
    col = lax.broadcasted_iota(jnp.int32, (rb, 128), 1)
    gidx = i * blk + row * 128 + col
    cv_ref[...] = jnp.where(gidx < n, cvec, 0.0)

  return pl.pallas_call(
      body,
      grid=grid,
      in_specs=[
          pl.BlockSpec((rb, 128), lambda i: (i, 0)),
          pl.BlockSpec((rb, 128), lambda i: (i, 0)),
          pl.BlockSpec((2, rb, 128), lambda i: (0, i, 0)),
          pl.BlockSpec((2, rb, 128), lambda i: (0, i, 0)),
      ],
      out_specs=[
          pl.BlockSpec((rb, 128), lambda i: (i, 0)),
          pl.BlockSpec((rb, 128), lambda i: (i, 0)),
      ],
      out_shape=[
          jax.ShapeDtypeStruct((r, 128), jnp.float32),
          jax.ShapeDtypeStruct((r, 128), jnp.float32),
      ],
  )(dinv2, y2, r1p, csp)


def _tc_expand(s1n, dinvn, a1, c1, W2, npad):
  blk = 2048
  grid = (npad // blk,)

  def body(s1_ref, dinv_ref, a1_ref, c1_ref, w2_ref,
           z0_ref, z1_ref, z2_ref, z3_ref, dg_ref):
    z_refs = [z0_ref, z1_ref, z2_ref, z3_ref]
    dinv = dinv_ref[...]
    h1 = jnp.maximum(s1_ref[...] * a1_ref[...] + c1_ref[...], 0.0)
    z = jnp.dot(h1, w2_ref[...], preferred_element_type=jnp.float32)
    zz = dinv * z
    for g in range(4):
      z_refs[g][...] = zz[:, 16 * g:16 * g + 16]
    dg_ref[...] = jnp.broadcast_to(dinv, (blk, 16))

  return pl.pallas_call(
      body,
      grid=grid,
      in_specs=[
          pl.BlockSpec((blk, 1), lambda i: (i, 0)),
          pl.BlockSpec((blk, 1), lambda i: (i, 0)),
          pl.BlockSpec((1, 64), lambda i: (0, 0)),
          pl.BlockSpec((1, 64), lambda i: (0, 0)),
          pl.BlockSpec((64, 64), lambda i: (0, 0)),
      ],
      out_specs=[pl.BlockSpec((blk, 16), lambda i: (i, 0))] * 5,
      out_shape=[jax.ShapeDtypeStruct((npad, 16), jnp.float32)
                 for _ in range(5)],
  )(s1n, dinvn, a1, c1, W2)


def _tc_final(r2g, zzg, dinvg, cvn, gs2, cb2, W3, a3, cc3,
              Wl1, bl1, Wl2, bl2, npad):
  blk = 2048
  grid = (npad // blk,)
  last = npad // blk - 1

  def body(*refs):
    r_refs = refs[:4]
    z_refs = refs[4:8]
    (dg_ref, cv_ref, gs2_ref, cb2_ref, w3_ref, a3_ref, cc3_ref,
     wl1_ref, bl1_ref, wl2_ref, bl2_ref, out_ref, acc_ref) = refs[8:]
    i = pl.program_id(0)

    @pl.when(i == 0)
    def _():
      acc_ref[...] = jnp.zeros_like(acc_ref)

    dg = dg_ref[...]
    cv = cv_ref[...]
    for g in range(4):
      out2 = dg * (r_refs[g][...] + z_refs[g][...])
      h2 = jnp.maximum(out2 * gs2_ref[0:1, 16 * g:16 * g + 16]
                       + cb2_ref[0:1, 16 * g:16 * g + 16], 0.0)
      pv = lax.dot_general(cv, h2, (((0,), (0,)), ((), ())),
                           preferred_element_type=jnp.float32)
      acc_ref[0:1, 16 * g:16 * g + 16] += pv

    @pl.when(i == last)
    def _():
      v = acc_ref[...]
      m = jnp.dot(v, w3_ref[...], preferred_element_type=jnp.float32)
      m = m * a3_ref[...] + cc3_ref[...]
      h = jnp.maximum(
          jnp.dot(m, wl1_ref[...], preferred_element_type=jnp.float32)
          + bl1_ref[...], 0.0)
      logits = jnp.dot(h, wl2_ref[...],
                       preferred_element_type=jnp.float32) + bl2_ref[...]
      mx = jnp.max(logits, axis=1, keepdims=True)
      ex = jnp.exp(logits - mx)
      out_ref[...] = ex / jnp.sum(ex, axis=1, keepdims=True)

  return pl.pallas_call(
      body,
      grid=grid,
      in_specs=[pl.BlockSpec((blk, 16), lambda i: (i, 0))] * 9
      + [
          pl.BlockSpec((blk, 1), lambda i: (i, 0)),
          pl.BlockSpec((1, 64), lambda i: (0, 0)),
          pl.BlockSpec((1, 64), lambda i: (0, 0)),
          pl.BlockSpec((64, 64), lambda i: (0, 0)),
          pl.BlockSpec((1, 64), lambda i: (0, 0)),
          pl.BlockSpec((1, 64), lambda i: (0, 0)),
          pl.BlockSpec((64, 32), lambda i: (0, 0)),
          pl.BlockSpec((1, 32), lambda i: (0, 0)),
          pl.BlockSpec((32, 3), lambda i: (0, 0)),
          pl.BlockSpec((1, 3), lambda i: (0, 0)),
      ],
      out_specs=pl.BlockSpec((1, 3), lambda i: (0, 0)),
      out_shape=jax.ShapeDtypeStruct((1, 3), jnp.float32),
      scratch_shapes=[pltpu.VMEM((1, 64), jnp.float32)],
  )(*r2g, *zzg, dinvg, cvn, gs2, cb2, W3, a3, cc3, Wl1, bl1, Wl2, bl2)


def kernel(x, edge_index, W1, b1, g1, be1, W2, b2, g2, be2, W3, b3, g3, be3,
           Wl1, bl1, Wl2, bl2):
  n = x.shape[0]
  e = edge_index.shape[1]
  npad, rows = _pad_sizes(n, e)
  epad = rows * 128

  xp = jnp.zeros((npad,), jnp.float32).at[:n].set(x[:, 0])
  pad_idx = (n + (jnp.arange(epad - e, dtype=jnp.int32) % 256))
  src = jnp.concatenate([edge_index[0].astype(jnp.int32), pad_idx])
  dst = jnp.concatenate([edge_index[1].astype(jnp.int32), pad_idx])
  src2d = src.reshape(rows, 128)
  dst2d = dst.reshape(rows, 128)
  xp2 = xp.reshape(npad // 128, 128)

  gs1 = g1 / jnp.sqrt(1.0 + _EPS)
  gs2 = g2 / jnp.sqrt(1.0 + _EPS)
  gs3 = g3 / jnp.sqrt(1.0 + _EPS)
  a1 = (W1[0] * gs1).reshape(1, 64)
  c1 = (b1 * gs1 + be1).reshape(1, 64)
  cb2 = (b2 * gs2 + be2).reshape(1, 64)
  a3 = (gs3 / np.float32(n)).reshape(1, 64)
  cc3 = (b3 * gs3 + be3).reshape(1, 64)

  degp = _sc_degree(dst2d, npad, rows)
  degp = degp.reshape(_NC, npad // 128, 128)
  dinv2, y2 = _tc_prep(degp, xp2, npad)

  r1p, csp = _sc_scalar_pass(src2d, dst2d, y2.reshape(npad),
                             dinv2.reshape(npad), npad, rows)
  r1p = r1p.reshape(_NC, npad // 128, 128)
  csp = csp.reshape(_NC, npad // 128, 128)
  s1_2d, cv2 = _tc_s1(dinv2, y2, r1p, csp, n, npad)
  dinvn = dinv2.reshape(npad, 1)
  exp_out = _tc_expand(s1_2d.reshape(npad, 1), dinvn, a1, c1, W2, npad)
  zzg, dinvg = exp_out[:4], exp_out[4]

  il2 = jnp.stack([src2d, dst2d], axis=1)
  zeros16 = jnp.zeros((128, 16), jnp.float32)
  r2g_a = _sc_wide_pass(il2, zzg[0:2], zeros16, npad, rows)
  r2g_b = _sc_wide_pass(il2, zzg[2:4], zeros16, npad, rows)
  r2g = [r2g_a[0], r2g_a[1], r2g_b[0], r2g_b[1]]
  out = _tc_final(r2g, zzg, dinvg, cv2.reshape(npad, 1),
                  gs2.reshape(1, 64), cb2, W3, a3, cc3, Wl1,
                  bl1.reshape(1, 32), Wl2.reshape(32, 3),
                  bl2.reshape(1, 3), npad)
  return out
